# Initial kernel scaffold; baseline (speedup 1.0000x reference)
import jax
import jax.numpy as jnp
from jax.experimental import pallas as pl


def kernel(input_ids, token_type_ids, token_w, pos_w, type_w, ln_w, ln_b):
    raise NotImplementedError("write your pallas kernel here")


# speedup vs baseline: 1.2735x; 1.2735x over previous
import functools

import jax
import jax.numpy as jnp
from jax import lax
from jax.experimental import pallas as pl
from jax.experimental.pallas import tpu as pltpu
from jax.experimental.pallas import tpu_sc as plsc

L = 16
NC, NS = 2, 16
NW = NC * NS


def _rsqrt(x):
    i = plsc.bitcast(x, jnp.int32)
    i = jnp.int32(0x5F3759DF) - (i >> 1)
    y = plsc.bitcast(i, jnp.float32)
    xh = x * jnp.float32(0.5)
    for _ in range(3):
        y = y * (jnp.float32(1.5) - xh * y * y)
    return y


def _make_kernel(B, T, H, CH):
    CT = T // CH
    NV = H // L
    NG = (T + L - 1) // L
    TP = NG * L
    seq_per_w = B // NW
    mesh = plsc.VectorSubcoreMesh(core_axis_name="c", subcore_axis_name="s")

    @functools.partial(
        pl.kernel,
        mesh=mesh,
        out_type=jax.ShapeDtypeStruct((B, T, H), jnp.float32),
        compiler_params=pltpu.CompilerParams(
            needs_layout_passes=False, use_tc_tiling_on_sc=False),
        scratch_types=[
            pltpu.VMEM((TP, H), jnp.float32),
            pltpu.VMEM((2, H), jnp.float32),
            pltpu.VMEM((H,), jnp.float32),
            pltpu.VMEM((H,), jnp.float32),
            pltpu.VMEM((H,), jnp.float32),
            pltpu.VMEM((CH, CT), jnp.int32),
            pltpu.VMEM((TP,), jnp.int32),
            pltpu.VMEM((TP, H), jnp.float32),
            pltpu.SemaphoreType.DMA,
            pltpu.SemaphoreType.DMA,
        ],
    )
    def emb(ids_hbm, tt_hbm, tok_hbm, pos_hbm, type_hbm, lnw_hbm, lnb_hbm,
            out_hbm, pos2_v, type_v, diff_v, lnw_v, lnb_v, idx_v, ttv_v,
            rows_v, gsem, osem):
        wid = lax.axis_index("s") * NC + lax.axis_index("c")

        pltpu.sync_copy(pos_hbm.at[pl.ds(0, T)], pos2_v.at[pl.ds(0, T)])
        pltpu.sync_copy(type_hbm, type_v)
        pltpu.sync_copy(lnw_hbm, lnw_v)
        pltpu.sync_copy(lnb_hbm, lnb_v)

        for k in range(NV):
            sl = pl.ds(k * L, L)
            diff_v[sl] = type_v[1, sl] - type_v[0, sl]

        def pos_body(t, carry):
            for k in range(NV):
                sl = pl.ds(k * L, L)
                pos2_v[t, sl] = pos2_v[t, sl] + type_v[0, sl]
            return carry

        lax.fori_loop(0, T, pos_body, 0)

        def pad_body(t, carry):
            for k in range(NV):
                sl = pl.ds(k * L, L)
                rows_v[t, sl] = jnp.zeros((L,), jnp.float32)
                pos2_v[t, sl] = jnp.zeros((L,), jnp.float32)
            return carry

        lax.fori_loop(T, TP, pad_body, 0)

        inv_h = jnp.float32(1.0 / H)
        eps = jnp.float32(1e-5)

        def seq_body(s, carry):
            seq = wid * seq_per_w + s
            pltpu.sync_copy(ids_hbm.at[seq], idx_v)
            pltpu.sync_copy(tt_hbm.at[seq], ttv_v.at[pl.ds(0, T)])
            copies = [
                pltpu.async_copy(
                    tok_hbm.at[idx_v.at[j]],
                    rows_v.at[pl.ds(j * CT, CT)],
                    gsem,
                )
                for j in range(CH)
            ]
            for cp in copies:
                cp.wait()

            def grp_body(g, carry):
                base = g * L
                ttf_vec = ttv_v[pl.ds(base, L)].astype(jnp.float32)
                for i in range(L):
                    t = base + i
                    ttf = ttf_vec[i]
                    vs = []
                    for k in range(NV):
                        sl = pl.ds(k * L, L)
                        v = rows_v[t, sl] + pos2_v[t, sl] + ttf * diff_v[sl]
                        vs.append(v)
                    acc = vs[0]
                    acc2 = vs[0] * vs[0]
                    for k in range(1, NV):
                        acc = acc + vs[k]
                        acc2 = acc2 + vs[k] * vs[k]
                    total = jnp.sum(acc)
                    total2 = jnp.sum(acc2)
                    mean = total * inv_h
                    var = total2 * inv_h - mean * mean
                    mean_v = jnp.full((L,), mean, dtype=jnp.float32)
                    inv_v = _rsqrt(
                        jnp.full((L,), var + eps, dtype=jnp.float32))
                    for k in range(NV):
                        sl = pl.ds(k * L, L)
                        rows_v[t, sl] = (
                            (vs[k] - mean_v) * inv_v * lnw_v[sl] + lnb_v[sl]
                        )
                return carry

            lax.fori_loop(0, NG, grp_body, 0)
            pltpu.sync_copy(rows_v.at[pl.ds(0, T)], out_hbm.at[seq])
            return carry

        lax.fori_loop(0, seq_per_w, seq_body, 0)

    return emb


def kernel(input_ids, token_type_ids, token_w, pos_w, type_w, ln_w, ln_b):
    B, T = input_ids.shape
    H = token_w.shape[1]
    CH = 2
    ids3 = input_ids.reshape(B, CH, T // CH)
    emb = _make_kernel(B, T, H, CH)
    return emb(ids3, token_type_ids, token_w, pos_w, type_w, ln_w, ln_b)


# speedup vs baseline: 2.5234x; 1.9815x over previous
import functools

import jax
import jax.numpy as jnp
from jax import lax
from jax.experimental import pallas as pl
from jax.experimental.pallas import tpu as pltpu
from jax.experimental.pallas import tpu_sc as plsc

L = 16
NC, NS = 2, 16
NW = NC * NS
NBUF = 3


def _rsqrt(x):
    i = plsc.bitcast(x, jnp.int32)
    i = jnp.int32(0x5F3759DF) - (i >> 1)
    y = plsc.bitcast(i, jnp.float32)
    xh = x * jnp.float32(0.5)
    for _ in range(3):
        y = y * (jnp.float32(1.5) - xh * y * y)
    return y


def _make_kernel(B, T, H, CH):
    CT = T // CH
    NV = H // L
    NG = (T + L - 1) // L
    TP = NG * L
    SW = B // NW
    mesh = plsc.VectorSubcoreMesh(core_axis_name="c", subcore_axis_name="s")

    @functools.partial(
        pl.kernel,
        mesh=mesh,
        out_type=jax.ShapeDtypeStruct((B, T, H), jnp.float32),
        compiler_params=pltpu.CompilerParams(
            needs_layout_passes=False, use_tc_tiling_on_sc=False),
        scratch_types=[
            pltpu.VMEM((TP, H), jnp.float32),
            pltpu.VMEM((2, H), jnp.float32),
            pltpu.VMEM((H,), jnp.float32),
            pltpu.VMEM((H,), jnp.float32),
            pltpu.VMEM((H,), jnp.float32),
            pltpu.VMEM((B // NW, CH, CT), jnp.int32),
            pltpu.VMEM((B // NW * T + L,), jnp.int32),
            pltpu.VMEM((NBUF, TP, H), jnp.float32),
            pltpu.VMEM((L, L + 1), jnp.float32),
            pltpu.VMEM((L, L + 1), jnp.float32),
            pltpu.SemaphoreType.DMA,
            pltpu.SemaphoreType.DMA((NBUF,)),
        ],
    )
    def emb(ids_hbm, tt_hbm, tok_hbm, pos_hbm, type_hbm, lnw_hbm, lnb_hbm,
            out_hbm, pos2_v, type_v, diff_v, lnw_v, lnb_v, ids_v, ttv_v,
            rows_v, sbuf_v, qbuf_v, gsem, osem):
        wid = lax.axis_index("s") * NC + lax.axis_index("c")

        pltpu.sync_copy(pos_hbm.at[pl.ds(0, T)], pos2_v.at[pl.ds(0, T)])
        pltpu.sync_copy(type_hbm, type_v)
        pltpu.sync_copy(lnw_hbm, lnw_v)
        pltpu.sync_copy(lnb_hbm, lnb_v)
        pltpu.sync_copy(ids_hbm.at[pl.ds(wid * SW, SW)], ids_v)
        pltpu.sync_copy(
            tt_hbm.at[pl.ds(wid * SW * T, SW * T)],
            ttv_v.at[pl.ds(0, SW * T)],
        )

        for k in range(NV):
            sl = pl.ds(k * L, L)
            diff_v[sl] = type_v[1, sl] - type_v[0, sl]

        def pos_body(t, carry):
            for k in range(NV):
                sl = pl.ds(k * L, L)
                pos2_v[t, sl] = pos2_v[t, sl] + type_v[0, sl]
            return carry

        lax.fori_loop(0, T, pos_body, 0)

        def pad_body(t, carry):
            for k in range(NV):
                sl = pl.ds(k * L, L)
                pos2_v[t, sl] = jnp.zeros((L,), jnp.float32)
                for nb in range(NBUF):
                    rows_v[nb, t, sl] = jnp.zeros((L,), jnp.float32)
            return carry

        lax.fori_loop(T, TP, pad_body, 0)

        inv_h = jnp.float32(1.0 / H)
        eps = jnp.float32(1e-5)
        ridx = lax.iota(jnp.int32, L)

        def issue_gathers(s, b):
            for j in range(CH):
                pltpu.async_copy(
                    tok_hbm.at[ids_v.at[s, j]],
                    rows_v.at[b, pl.ds(j * CT, CT)],
                    gsem,
                )

        def wait_gathers(s, b):
            for j in range(CH):
                pltpu.make_async_copy(
                    tok_hbm.at[ids_v.at[s, j]],
                    rows_v.at[b, pl.ds(j * CT, CT)],
                    gsem,
                ).wait()

        def wait_out(b, seq):
            pltpu.make_async_copy(
                rows_v.at[b, pl.ds(0, T)], out_hbm.at[seq], osem.at[b]
            ).wait()

        issue_gathers(0, 0)

        def seq_body(s, carry):
            seq = wid * SW + s
            b = lax.rem(s, NBUF)
            bn = lax.rem(s + 1, NBUF)
            wait_gathers(s, b)

            @pl.when(s < SW - 1)
            def _():
                @pl.when(s >= NBUF - 1)
                def _():
                    wait_out(bn, seq)
                issue_gathers(s + 1, bn)

            def grp_body(g, carry):
                base = g * L
                tbase = s * T + base
                ttf_vec = ttv_v[pl.ds(tbase, L)].astype(jnp.float32)
                dk = [diff_v[pl.ds(k * L, L)] for k in range(NV)]
                for i in range(L):
                    t = base + i
                    ttf = ttf_vec[i]
                    vs = []
                    for k in range(NV):
                        sl = pl.ds(k * L, L)
                        v = rows_v[b, t, sl] + (pos2_v[t, sl] + ttf * dk[k])
                        vs.append(v)
                    s0 = (vs[0] + vs[1]) + (vs[2] + vs[3])
                    s1 = (vs[4] + vs[5]) + (vs[6] + vs[7])
                    q0 = (vs[0] * vs[0] + vs[1] * vs[1]) + (
                        vs[2] * vs[2] + vs[3] * vs[3])
                    q1 = (vs[4] * vs[4] + vs[5] * vs[5]) + (
                        vs[6] * vs[6] + vs[7] * vs[7])
                    sbuf_v[i, pl.ds(0, L)] = s0 + s1
                    qbuf_v[i, pl.ds(0, L)] = q0 + q1
                    for k in range(NV):
                        sl = pl.ds(k * L, L)
                        rows_v[b, t, sl] = vs[k]
                sums = plsc.load_gather(
                    sbuf_v, [ridx, jnp.zeros((L,), jnp.int32)])
                qsums = plsc.load_gather(
                    qbuf_v, [ridx, jnp.zeros((L,), jnp.int32)])
                for l in range(1, L):
                    cidx = jnp.full((L,), l, dtype=jnp.int32)
                    sums = sums + plsc.load_gather(sbuf_v, [ridx, cidx])
                    qsums = qsums + plsc.load_gather(qbuf_v, [ridx, cidx])
                mean16 = sums * inv_h
                var16 = qsums * inv_h - mean16 * mean16
                inv16 = _rsqrt(var16 + eps)
                beta16 = -mean16 * inv16
                wk = [lnw_v[pl.ds(k * L, L)] for k in range(NV)]
                bk = [lnb_v[pl.ds(k * L, L)] for k in range(NV)]
                for i in range(L):
                    t = base + i
                    r = inv16[i]
                    bb = beta16[i]
                    for k in range(NV):
                        sl = pl.ds(k * L, L)
                        y = rows_v[b, t, sl] * r + bb
                        rows_v[b, t, sl] = y * wk[k] + bk[k]
                return carry

            lax.fori_loop(0, NG, grp_body, 0)
            pltpu.async_copy(
                rows_v.at[b, pl.ds(0, T)], out_hbm.at[seq], osem.at[b])
            return carry

        lax.fori_loop(0, SW, seq_body, 0)

        for s in (SW - 3, SW - 2, SW - 1):
            wait_out(s % NBUF, wid * SW + s)

    return emb


def kernel(input_ids, token_type_ids, token_w, pos_w, type_w, ln_w, ln_b):
    B, T = input_ids.shape
    H = token_w.shape[1]
    CH = 2
    ids3 = input_ids.reshape(B, CH, T // CH)
    ttf = token_type_ids.reshape(B * T)
    emb = _make_kernel(B, T, H, CH)
    return emb(ids3, ttf, token_w, pos_w, type_w, ln_w, ln_b)


# speedup vs baseline: 5.9050x; 2.3401x over previous
import functools

import jax
import jax.numpy as jnp
from jax import lax
from jax.experimental import pallas as pl
from jax.experimental.pallas import tpu as pltpu
from jax.experimental.pallas import tpu_sc as plsc

L = 16
NC, NS = 2, 16
NW = NC * NS


def _rsqrt(x):
    i = plsc.bitcast(x, jnp.int32)
    i = jnp.int32(0x5F3759DF) - (i >> 1)
    y = plsc.bitcast(i, jnp.float32)
    xh = x * jnp.float32(0.5)
    for _ in range(3):
        y = y * (jnp.float32(1.5) - xh * y * y)
    return y


def _make_kernel(B, T, H, CH):
    CT = T // CH
    NV = H // L
    NG = (T + L - 1) // L
    TP = NG * L
    SW = B // NW
    mesh = plsc.VectorSubcoreMesh(core_axis_name="c", subcore_axis_name="s")

    @functools.partial(
        pl.kernel,
        mesh=mesh,
        out_type=jax.ShapeDtypeStruct((B, T, H), jnp.float32),
        compiler_params=pltpu.CompilerParams(
            needs_layout_passes=False, use_tc_tiling_on_sc=False),
        scratch_types=[
            pltpu.VMEM((TP, H), jnp.float32),
            pltpu.VMEM((2, H), jnp.float32),
            pltpu.VMEM((H,), jnp.float32),
            pltpu.VMEM((H,), jnp.float32),
            pltpu.VMEM((H,), jnp.float32),
            pltpu.VMEM((B // NW, CH, CT), jnp.int32),
            pltpu.VMEM((B // NW * T + L,), jnp.int32),
            pltpu.VMEM((TP, H), jnp.float32),
            pltpu.VMEM((TP, H), jnp.float32),
            pltpu.VMEM((L, L + 1), jnp.float32),
            pltpu.VMEM((L, L + 1), jnp.float32),
            pltpu.SemaphoreType.DMA,
            pltpu.SemaphoreType.DMA,
            pltpu.SemaphoreType.DMA,
        ],
    )
    def emb(ids_hbm, tt_hbm, tok_hbm, pos_hbm, type_hbm, lnw_hbm, lnb_hbm,
            out_hbm, pos2_v, type_v, diff_v, lnw_v, lnb_v, ids_v, ttv_v,
            rows0_v, rows1_v, sbuf_v, qbuf_v, gsem, osem0, osem1):
        wid = lax.axis_index("s") * NC + lax.axis_index("c")
        rows = (rows0_v, rows1_v)
        osems = (osem0, osem1)

        pltpu.sync_copy(pos_hbm.at[pl.ds(0, T)], pos2_v.at[pl.ds(0, T)])
        pltpu.sync_copy(type_hbm, type_v)
        pltpu.sync_copy(lnw_hbm, lnw_v)
        pltpu.sync_copy(lnb_hbm, lnb_v)
        pltpu.sync_copy(ids_hbm.at[pl.ds(wid * SW, SW)], ids_v)
        pltpu.sync_copy(
            tt_hbm.at[pl.ds(wid * SW * T, SW * T)],
            ttv_v.at[pl.ds(0, SW * T)],
        )

        for k in range(NV):
            sl = pl.ds(k * L, L)
            diff_v[sl] = type_v[1, sl] - type_v[0, sl]

        def pos_body(t, carry):
            for k in range(NV):
                sl = pl.ds(k * L, L)
                pos2_v[t, sl] = pos2_v[t, sl] + type_v[0, sl]
            return carry

        lax.fori_loop(0, T, pos_body, 0)

        def pad_body(t, carry):
            for k in range(NV):
                sl = pl.ds(k * L, L)
                pos2_v[t, sl] = jnp.zeros((L,), jnp.float32)
                rows0_v[t, sl] = jnp.zeros((L,), jnp.float32)
                rows1_v[t, sl] = jnp.zeros((L,), jnp.float32)
            return carry

        lax.fori_loop(T, TP, pad_body, 0)

        inv_h = jnp.float32(1.0 / H)
        eps = jnp.float32(1e-5)
        ridx = lax.iota(jnp.int32, L)

        def issue_gathers(s, rbuf):
            for j in range(CH):
                pltpu.async_copy(
                    tok_hbm.at[ids_v.at[s, j]],
                    rbuf.at[pl.ds(j * CT, CT)],
                    gsem,
                )

        def wait_gathers(s, rbuf):
            for j in range(CH):
                pltpu.make_async_copy(
                    tok_hbm.at[ids_v.at[s, j]],
                    rbuf.at[pl.ds(j * CT, CT)],
                    gsem,
                ).wait()

        def wait_out(rbuf, osem, seq):
            pltpu.make_async_copy(
                rbuf.at[pl.ds(0, T)], out_hbm.at[seq], osem
            ).wait()

        def compute_seq(s, rbuf):
            def grp_body(g, carry):
                base = g * L
                tbase = s * T + base
                ttf_vec = ttv_v[pl.ds(tbase, L)].astype(jnp.float32)
                dk = [diff_v[pl.ds(k * L, L)] for k in range(NV)]
                for i in range(L):
                    t = base + i
                    ttf = ttf_vec[i]
                    vs = []
                    for k in range(NV):
                        sl = pl.ds(k * L, L)
                        v = rbuf[t, sl] + (pos2_v[t, sl] + ttf * dk[k])
                        vs.append(v)
                    s0 = (vs[0] + vs[1]) + (vs[2] + vs[3])
                    s1 = (vs[4] + vs[5]) + (vs[6] + vs[7])
                    q0 = (vs[0] * vs[0] + vs[1] * vs[1]) + (
                        vs[2] * vs[2] + vs[3] * vs[3])
                    q1 = (vs[4] * vs[4] + vs[5] * vs[5]) + (
                        vs[6] * vs[6] + vs[7] * vs[7])
                    sbuf_v[i, pl.ds(0, L)] = s0 + s1
                    qbuf_v[i, pl.ds(0, L)] = q0 + q1
                    for k in range(NV):
                        sl = pl.ds(k * L, L)
                        rbuf[t, sl] = vs[k]
                sums = plsc.load_gather(
                    sbuf_v, [ridx, jnp.zeros((L,), jnp.int32)])
                qsums = plsc.load_gather(
                    qbuf_v, [ridx, jnp.zeros((L,), jnp.int32)])
                for l in range(1, L):
                    cidx = jnp.full((L,), l, dtype=jnp.int32)
                    sums = sums + plsc.load_gather(sbuf_v, [ridx, cidx])
                    qsums = qsums + plsc.load_gather(qbuf_v, [ridx, cidx])
                mean16 = sums * inv_h
                var16 = qsums * inv_h - mean16 * mean16
                inv16 = _rsqrt(var16 + eps)
                beta16 = -mean16 * inv16
                wk = [lnw_v[pl.ds(k * L, L)] for k in range(NV)]
                bk = [lnb_v[pl.ds(k * L, L)] for k in range(NV)]
                for i in range(L):
                    t = base + i
                    r = inv16[i]
                    bb = beta16[i]
                    for k in range(NV):
                        sl = pl.ds(k * L, L)
                        y = rbuf[t, sl] * r + bb
                        rbuf[t, sl] = y * wk[k] + bk[k]
                return carry

            lax.fori_loop(0, NG, grp_body, 0)

        issue_gathers(0, rows[0])

        def pair_body(p, carry):
            for half in range(2):
                s = p * 2 + half
                seq = wid * SW + s
                cur, nxt = rows[half], rows[1 - half]
                wait_gathers(s, cur)

                @pl.when(s < SW - 1)
                def _():
                    @pl.when(s >= 1)
                    def _():
                        wait_out(nxt, osems[1 - half], seq)
                    issue_gathers(s + 1, nxt)

                compute_seq(s, cur)
                pltpu.async_copy(
                    cur.at[pl.ds(0, T)], out_hbm.at[seq], osems[half])
            return carry

        lax.fori_loop(0, SW // 2, pair_body, 0)

        wait_out(rows[0], osems[0], wid * SW + SW - 2)
        wait_out(rows[1], osems[1], wid * SW + SW - 1)

    return emb


def kernel(input_ids, token_type_ids, token_w, pos_w, type_w, ln_w, ln_b):
    B, T = input_ids.shape
    H = token_w.shape[1]
    CH = 2
    ids3 = input_ids.reshape(B, CH, T // CH)
    ttf = token_type_ids.reshape(B * T)
    emb = _make_kernel(B, T, H, CH)
    return emb(ids3, ttf, token_w, pos_w, type_w, ln_w, ln_b)
